# baseline (speedup 1.0000x reference)
import jax
import jax.numpy as jnp
from jax.experimental import pallas as pl

_G = 8
_V = 25
_VIRT = 5
_VP = _V + _VIRT
_K = 9
_C = 256
_T = 64
_N = 32
_CG = _C // _G
_HID = 8


def _bdot(a, b, ca, cb):
    return jax.lax.dot_general(
        a, b, (((ca,), (cb,)), ((0,), (0,))),
        preferred_element_type=jnp.float32)


def _main_kernel(x_ref, hj_ref, vw_ref, vb_ref, w0w_ref, w0b_ref,
                 w2w_ref, w2b_ref, alpha_ref, dw_ref, db_ref,
                 pa_ref, ei_ref, y_ref, s1_ref, s2_ref):
    x = x_ref[0]
    hj = hj_ref[...]
    hx = jnp.broadcast_to(hj[:, None, :], (_C, _T, _VIRT))
    xf = jnp.concatenate([x, hx], axis=-1)

    tx = jnp.mean(xf, axis=1)
    txg = tx.reshape(_G, _CG, _VP)

    v = _bdot(vw_ref[...], txg, 2, 1) + vb_ref[...][:, :, None]
    sq = jnp.sum(v * v, axis=1)
    cross = _bdot(v, v, 1, 1)
    d2 = sq[:, :, None] + sq[:, None, :] - 2.0 * cross
    D = jnp.sqrt(jnp.maximum(d2, 1e-12))

    a4 = D[:, :, :, None]
    b4 = D[:, :, None, :]
    less = jnp.sum((b4 < a4).astype(jnp.float32), axis=-1)
    wi = jax.lax.broadcasted_iota(jnp.int32, (_G, _VP, _VP, _VP), 3)
    vi = jax.lax.broadcasted_iota(jnp.int32, (_G, _VP, _VP, _VP), 2)
    eqb = jnp.sum(jnp.where((b4 == a4) & (wi < vi), 1.0, 0.0), axis=-1)
    sel = (less + eqb) < float(_K)

    neg = -D
    mx = jnp.max(neg, axis=-1, keepdims=True)
    e = jnp.where(sel, jnp.exp(neg - mx), 0.0)
    H = e / jnp.sum(e, axis=-1, keepdims=True)

    w1 = _bdot(w0w_ref[...], txg, 2, 1) + w0b_ref[...][:, :, None]
    w1 = jnp.where(w1 >= 0.0, w1, 0.01 * w1)
    Wm = jnp.tanh(jnp.dot(w2w_ref[...], w1.reshape(_G * _HID, _VP),
                          preferred_element_type=jnp.float32)
                  + w2b_ref[...])

    norm_w = jnp.sum(jnp.abs(H), axis=1) + 1e-8
    Hw = H * Wm[:, None, :]
    norm_v = jnp.sum(jnp.abs(Hw), axis=2, keepdims=True) + 1e-8
    M = (Hw / norm_v) * (Wm / norm_w)[:, None, :]
    Hn = _bdot(M[:, :_V, :], H, 2, 2)

    a0 = ei_ref[...] * pa_ref[...]
    a0 = a0 / (jnp.abs(a0) + 1e-8)
    al = jnp.maximum(alpha_ref[0, 0], 0.0)
    A25 = a0[:, :, None] + al * Hn

    xfg = xf.reshape(_G, _CG, _T, _VP)
    dx = _bdot(dw_ref[...], xfg, 2, 1) + db_ref[...][:, :, None, None]
    y = _bdot(dx, A25, 3, 2)
    yf = y.reshape(_C, _T, _V)

    y_ref[...] = yf[None]
    s1_ref[...] = jnp.sum(yf, axis=(1, 2)).reshape(1, 1, _C)
    s2_ref[...] = jnp.sum(yf * yf, axis=(1, 2)).reshape(1, 1, _C)


def _bn_kernel(y_ref, x_ref, s1_ref, s2_ref, bnw_ref, bnb_ref, o_ref):
    cnt = float(_N * _T * _V)
    mean = jnp.sum(s1_ref[...], axis=0)[0] / cnt
    ey2 = jnp.sum(s2_ref[...], axis=0)[0] / cnt
    var = ey2 - mean * mean
    scale = bnw_ref[0] * jax.lax.rsqrt(var + 1e-5)
    shift = bnb_ref[0] - mean * scale
    y = y_ref[0]
    out = y * scale[:, None, None] + shift[:, None, None] + x_ref[0]
    o_ref[...] = jnp.maximum(out, 0.0)[None]


def kernel(x, hyper_joint, to_V_w, to_V_b, to_W0_w, to_W0_b, to_W2_w,
           to_W2_b, alpha, conv_d_w, conv_d_b, PA, edge_importance,
           bn_w, bn_b):
    f32 = jnp.float32
    hjT = hyper_joint.T
    vw = to_V_w[:, :, 0].reshape(_G, _HID, _CG)
    vb = to_V_b.reshape(_G, _HID)
    w0w = to_W0_w[:, :, 0].reshape(_G, _HID, _CG)
    w0b = to_W0_b.reshape(_G, _HID)
    w2w = to_W2_w[:, :, 0]
    w2b = to_W2_b.reshape(_G, 1)
    dw = conv_d_w[:, :, 0, 0].reshape(_G, _CG, _CG)
    db = conv_d_b.reshape(_G, _CG)
    pa = PA.reshape(_G, 1)
    ei = edge_importance.reshape(_G, 1)
    al = alpha.reshape(1, 1)

    rep = lambda s: pl.BlockSpec(s, lambda n: (0,) * len(s))
    per_n4 = pl.BlockSpec((1, _C, _T, _V), lambda n: (n, 0, 0, 0))
    per_n3 = pl.BlockSpec((1, 1, _C), lambda n: (n, 0, 0))

    y_pre, s1, s2 = pl.pallas_call(
        _main_kernel,
        grid=(_N,),
        in_specs=[
            per_n4,
            rep((_C, _VIRT)),
            rep((_G, _HID, _CG)),
            rep((_G, _HID)),
            rep((_G, _HID, _CG)),
            rep((_G, _HID)),
            rep((_G, _G * _HID)),
            rep((_G, 1)),
            rep((1, 1)),
            rep((_G, _CG, _CG)),
            rep((_G, _CG)),
            rep((_G, 1)),
            rep((_G, 1)),
        ],
        out_specs=[per_n4, per_n3, per_n3],
        out_shape=[
            jax.ShapeDtypeStruct((_N, _C, _T, _V), f32),
            jax.ShapeDtypeStruct((_N, 1, _C), f32),
            jax.ShapeDtypeStruct((_N, 1, _C), f32),
        ],
    )(x, hjT, vw, vb, w0w, w0b, w2w, w2b, al, dw, db, pa, ei)

    out = pl.pallas_call(
        _bn_kernel,
        grid=(_N,),
        in_specs=[
            per_n4,
            per_n4,
            rep((_N, 1, _C)),
            rep((_N, 1, _C)),
            rep((1, _C)),
            rep((1, _C)),
        ],
        out_specs=per_n4,
        out_shape=jax.ShapeDtypeStruct((_N, _C, _T, _V), f32),
    )(y_pre, x, s1, s2, bn_w.reshape(1, _C), bn_b.reshape(1, _C))

    return (out, hyper_joint)
